# baseline (speedup 1.0000x reference)
import functools

import jax
import jax.numpy as jnp
from jax import lax
from jax.experimental import pallas as pl
from jax.experimental.pallas import tpu as pltpu
from jax.experimental.pallas import tpu_sc as plsc

N = 10000
E = 320000
D = 128
N_P = 10240
PAD_ROW = 10000
NC, NS = 2, 16
NW = NC * NS
E_W = E // NW
CH = 64
NCH = 160
NBUF = 4
RS = 8
E_PAD = NW * NCH * CH
ROWS_T = N_P // NS
LAYERS = (128, 128, 128, 128, 128, 64)
BLK = 1280


def _spmm_body(idx_hbm, s_hbm, out_hbm,
               idxr, rows, acc_sh, semi, semg):
    c = lax.axis_index("c")
    sid = lax.axis_index("s")
    wid = c * NS + sid

    zv = jnp.zeros((16,), jnp.float32)

    def zrow(r, carry):
        for k in range(D // 16):
            rows[0, r, pl.ds(k * 16, 16)] = zv
        return carry

    lax.fori_loop(0, CH, zrow, 0, unroll=False)

    def zcopy(k, carry):
        pltpu.sync_copy(rows.at[0],
                        acc_sh.at[pl.ds(sid * ROWS_T + k * CH, CH)])
        return carry

    lax.fori_loop(0, ROWS_T // CH, zcopy, 0, unroll=False)
    plsc.subcore_barrier()

    def idx_dma(j, slot):
        return pltpu.make_async_copy(idx_hbm.at[wid].at[j], idxr.at[slot],
                                     semi.at[slot])

    def gather(slot, b):
        return pltpu.make_async_copy(s_hbm.at[idxr.at[slot, 0]], rows.at[b],
                                     semg.at[b])

    for slot in range(RS):
        idx_dma(slot, slot).start()
    for b in range(NBUF):
        idx_dma(b, b).wait()
        gather(b, b).start()

    def body(jr, carry):
        for b in range(RS):
            j = RS * jr + b
            q = b % NBUF
            gather(b, q).wait()
            pltpu.sync_copy(rows.at[q], acc_sh.at[idxr.at[b, 1]], add=True)

            @pl.when(j + RS < NCH)
            def _():
                idx_dma(j + RS, b).start()

            @pl.when(j + NBUF < NCH)
            def _():
                idx_dma(j + NBUF, (b + NBUF) % RS).wait()
                gather((b + NBUF) % RS, q).start()
        return carry

    lax.fori_loop(0, NCH // RS, body, 0, unroll=False)

    plsc.subcore_barrier()
    pltpu.sync_copy(acc_sh.at[pl.ds(sid * ROWS_T, ROWS_T)],
                    out_hbm.at[pl.ds((c * N_P + sid * ROWS_T), ROWS_T)])


_spmm = pl.kernel(
    _spmm_body,
    out_type=jax.ShapeDtypeStruct((NC * N_P, D), jnp.float32),
    mesh=plsc.VectorSubcoreMesh(core_axis_name="c", subcore_axis_name="s",
                                num_cores=NC, num_subcores=NS),
    scratch_types=[
        pltpu.VMEM((RS, 2, CH), jnp.int32),
        pltpu.VMEM((NBUF, CH, D), jnp.float32),
        pltpu.VMEM_SHARED((N_P, D), jnp.float32),
        pltpu.SemaphoreType.DMA((RS,)),
        pltpu.SemaphoreType.DMA((NBUF,)),
    ],
    name="sc_spmm",
)


def _deg_body(idx_hbm, out_hbm, idxr, rows, acc_sh, semi):
    c = lax.axis_index("c")
    sid = lax.axis_index("s")
    wid = c * NS + sid

    zv = jnp.zeros((16,), jnp.float32)
    ov = jnp.ones((16,), jnp.float32)

    def frow(r, carry):
        for k in range(D // 16):
            rows[0, r, pl.ds(k * 16, 16)] = zv
            rows[1, r, pl.ds(k * 16, 16)] = ov
        return carry

    lax.fori_loop(0, CH, frow, 0, unroll=False)

    def zcopy(k, carry):
        pltpu.sync_copy(rows.at[0],
                        acc_sh.at[pl.ds(sid * ROWS_T + k * CH, CH)])
        return carry

    lax.fori_loop(0, ROWS_T // CH, zcopy, 0, unroll=False)
    plsc.subcore_barrier()

    def idx_dma(j, slot):
        return pltpu.make_async_copy(idx_hbm.at[wid].at[j], idxr.at[slot],
                                     semi.at[slot])

    for slot in range(RS):
        idx_dma(slot, slot).start()

    def body(jr, carry):
        for b in range(RS):
            j = RS * jr + b
            idx_dma(j, b).wait()
            pltpu.sync_copy(rows.at[1], acc_sh.at[idxr.at[b, 1]], add=True)

            @pl.when(j + RS < NCH)
            def _():
                idx_dma(j + RS, b).start()
        return carry

    lax.fori_loop(0, NCH // RS, body, 0, unroll=False)

    plsc.subcore_barrier()
    pltpu.sync_copy(acc_sh.at[pl.ds(sid * ROWS_T, ROWS_T)],
                    out_hbm.at[pl.ds((c * N_P + sid * ROWS_T), ROWS_T)])


_deg = pl.kernel(
    _deg_body,
    out_type=jax.ShapeDtypeStruct((NC * N_P, D), jnp.float32),
    mesh=plsc.VectorSubcoreMesh(core_axis_name="c", subcore_axis_name="s",
                                num_cores=NC, num_subcores=NS),
    scratch_types=[
        pltpu.VMEM((RS, 2, CH), jnp.int32),
        pltpu.VMEM((2, CH, D), jnp.float32),
        pltpu.VMEM_SHARED((N_P, D), jnp.float32),
        pltpu.SemaphoreType.DMA((RS,)),
    ],
    name="sc_deg",
)


def _prep_body(degp_ref, x_ref, dinv_ref, s0_ref):
    deg = degp_ref[0] + degp_ref[1]
    dinv = lax.rsqrt(jnp.maximum(deg, 1.0))
    dinv_ref[...] = dinv
    s0_ref[...] = dinv * x_ref[...]


def _prep(degp, x_pad):
    grid = N_P // BLK
    return pl.pallas_call(
        _prep_body,
        grid=(grid,),
        in_specs=[
            pl.BlockSpec((2, BLK, D), lambda b: (0, b, 0)),
            pl.BlockSpec((BLK, D), lambda b: (b, 0)),
        ],
        out_specs=[
            pl.BlockSpec((BLK, D), lambda b: (b, 0)),
            pl.BlockSpec((BLK, D), lambda b: (b, 0)),
        ],
        out_shape=[
            jax.ShapeDtypeStruct((N_P, D), jnp.float32),
            jax.ShapeDtypeStruct((N_P, D), jnp.float32),
        ],
    )(degp, x_pad)


def _mid_body(p_ref, dinv_ref, s1_ref):
    dinv = dinv_ref[...]
    s1_ref[...] = dinv * dinv * (p_ref[0] + p_ref[1])


def _mid(p, dinv):
    grid = N_P // BLK
    return pl.pallas_call(
        _mid_body,
        grid=(grid,),
        in_specs=[
            pl.BlockSpec((2, BLK, D), lambda b: (0, b, 0)),
            pl.BlockSpec((BLK, D), lambda b: (b, 0)),
        ],
        out_specs=pl.BlockSpec((BLK, D), lambda b: (b, 0)),
        out_shape=jax.ShapeDtypeStruct((N_P, D), jnp.float32),
    )(p, dinv)


def _end_body(h_ref, s1_ref, p2_ref, dinv_ref, w_ref, g_ref, b_ref,
              skip_ref, h_new_ref, s_next_ref, *, od, has_skip):
    dinv = dinv_ref[...]
    f1 = s1_ref[...] / dinv
    f2 = dinv * (p2_ref[0] + p2_ref[1])
    w = w_ref[...]
    pre = jnp.dot(h_ref[...], w[:D], preferred_element_type=jnp.float32)
    pre += jnp.dot(f1, w[D:2 * D], preferred_element_type=jnp.float32)
    pre += jnp.dot(f2, w[2 * D:], preferred_element_type=jnp.float32)
    h_new = jnp.maximum(g_ref[...] * pre + b_ref[...], 0.0)
    if has_skip:
        h_new = h_new + skip_ref[...]
    h_new_ref[...] = h_new
    s_next_ref[...] = dinv[:, :od] * h_new


def _end(h, s1, p2, dinv, w, gamma, beta, skip):
    od = w.shape[1]
    has_skip = skip is not None
    if skip is None:
        skip = jnp.zeros((1, 1), jnp.float32)
        skip_spec = pl.BlockSpec((1, 1), lambda b: (0, 0))
    else:
        skip_spec = pl.BlockSpec((BLK, od), lambda b: (b, 0))
    grid = N_P // BLK
    body = functools.partial(_end_body, od=od, has_skip=has_skip)
    return pl.pallas_call(
        body,
        grid=(grid,),
        in_specs=[
            pl.BlockSpec((BLK, D), lambda b: (b, 0)),
            pl.BlockSpec((BLK, D), lambda b: (b, 0)),
            pl.BlockSpec((2, BLK, D), lambda b: (0, b, 0)),
            pl.BlockSpec((BLK, D), lambda b: (b, 0)),
            pl.BlockSpec((3 * D, od), lambda b: (0, 0)),
            pl.BlockSpec((1, od), lambda b: (0, 0)),
            pl.BlockSpec((1, od), lambda b: (0, 0)),
            skip_spec,
        ],
        out_specs=[
            pl.BlockSpec((BLK, od), lambda b: (b, 0)),
            pl.BlockSpec((BLK, od), lambda b: (b, 0)),
        ],
        out_shape=[
            jax.ShapeDtypeStruct((N_P, od), jnp.float32),
            jax.ShapeDtypeStruct((N_P, od), jnp.float32),
        ],
    )(h, s1, p2, dinv, w, gamma.reshape(1, od), beta.reshape(1, od), skip)


def _pool_body(h_ref, ag_ref, ab_ref, wo_ref, bo_ref, out_ref):
    h = h_ref[...]
    rows = lax.broadcasted_iota(jnp.int32, h.shape, 0)
    valid = rows < N
    neg = jnp.float32(-3.4e38)
    pos = jnp.float32(3.4e38)
    mx = jnp.max(jnp.where(valid, h, neg), axis=0)
    mn = jnp.min(jnp.where(valid, h, pos), axis=0)
    hz = jnp.where(valid, h, 0.0)
    sm = jnp.sum(hz, axis=0)
    mean = sm / N
    cent = jnp.where(valid, h - mean[None, :], 0.0)
    var = jnp.sum(cent * cent, axis=0) / (N - 1)
    std = jnp.sqrt(var)
    flat = jnp.concatenate([mx, mn, mean, std], axis=0)
    z = flat * ag_ref[0] + ab_ref[0]
    res = jnp.sum(z * wo_ref[:, 0]) + bo_ref[0, 0]
    out_ref[...] = jnp.full((1, 128), res, jnp.float32)


def _pool(h5, agg_gamma, agg_beta, w_out, b_out):
    return pl.pallas_call(
        _pool_body,
        out_shape=jax.ShapeDtypeStruct((1, 128), jnp.float32),
    )(h5, agg_gamma.reshape(1, -1), agg_beta.reshape(1, -1),
      w_out, b_out.reshape(1, 1))


def kernel(x, edge_index, Ws, gammas, betas, agg_gamma, agg_beta, W_out, b_out):
    src = edge_index[0]
    dst = edge_index[1]
    pad = jnp.full((E_PAD - E,), PAD_ROW, jnp.int32)
    srcp = jnp.concatenate([src, pad]).reshape(NW, NCH, CH)
    dstp = jnp.concatenate([dst, pad]).reshape(NW, NCH, CH)
    ei = jnp.stack([srcp, dstp], axis=2)
    x_pad = jnp.pad(x, ((0, N_P - N), (0, 0)))

    spmm = lambda s: _spmm(ei, s).reshape(NC, N_P, D)

    degp = _deg(ei).reshape(NC, N_P, D)
    dinv, s = _prep(degp, x_pad)

    h = x_pad
    outs = []
    for i in range(len(LAYERS)):
        p1 = spmm(s)
        s1 = _mid(p1, dinv)
        p2 = spmm(s1)
        skip = outs[i - 2] if (i >= 2 and LAYERS[i] == LAYERS[i - 2]) else None
        h, s = _end(h, s1, p2, dinv, Ws[i], gammas[i], betas[i], skip)
        outs.append(h)

    out = _pool(h, agg_gamma, agg_beta, W_out, b_out)
    return out[0, :1]


# speedup vs baseline: 1.0616x; 1.0616x over previous
import functools

import jax
import jax.numpy as jnp
from jax import lax
from jax.experimental import pallas as pl
from jax.experimental.pallas import tpu as pltpu
from jax.experimental.pallas import tpu_sc as plsc

N = 10000
E = 320000
D = 128
N_P = 10240
PAD_ROW = 10000
NC, NS = 2, 16
NW = NC * NS
E_W = E // NW
CH = 64
NCH = 160
NBUF = 4
RS = 8
E_PAD = NW * NCH * CH
ROWS_T = N_P // NS
LAYERS = (128, 128, 128, 128, 128, 64)
BLK = 1280


def _spmm_body(idx_hbm, s_hbm, out_hbm,
               idxr, rows, acc_sh, semi, semg):
    c = lax.axis_index("c")
    sid = lax.axis_index("s")
    wid = c * NS + sid

    zv = jnp.zeros((16,), jnp.float32)

    def zrow(r, carry):
        for k in range(D // 16):
            rows[0, r, pl.ds(k * 16, 16)] = zv
        return carry

    lax.fori_loop(0, CH, zrow, 0, unroll=False)

    def zcopy(k, carry):
        pltpu.sync_copy(rows.at[0],
                        acc_sh.at[pl.ds(sid * ROWS_T + k * CH, CH)])
        return carry

    lax.fori_loop(0, ROWS_T // CH, zcopy, 0, unroll=False)
    plsc.subcore_barrier()

    def idx_dma(j, slot):
        return pltpu.make_async_copy(idx_hbm.at[wid].at[j], idxr.at[slot],
                                     semi.at[slot])

    def gather(slot, b):
        return pltpu.make_async_copy(s_hbm.at[idxr.at[slot, 0]], rows.at[b],
                                     semg.at[b])

    for slot in range(RS):
        idx_dma(slot, slot).start()
    for b in range(NBUF):
        idx_dma(b, b).wait()
        gather(b, b).start()

    def body(jr, carry):
        for b in range(RS):
            j = RS * jr + b
            q = b % NBUF
            gather(b, q).wait()
            pltpu.sync_copy(rows.at[q], acc_sh.at[idxr.at[b, 1]], add=True)

            @pl.when(j + RS < NCH)
            def _():
                idx_dma(j + RS, b).start()

            @pl.when(j + NBUF < NCH)
            def _():
                idx_dma(j + NBUF, (b + NBUF) % RS).wait()
                gather((b + NBUF) % RS, q).start()
        return carry

    lax.fori_loop(0, NCH // RS, body, 0, unroll=False)

    plsc.subcore_barrier()
    pltpu.sync_copy(acc_sh.at[pl.ds(sid * ROWS_T, ROWS_T)],
                    out_hbm.at[pl.ds((c * N_P + sid * ROWS_T), ROWS_T)])


_spmm = pl.kernel(
    _spmm_body,
    out_type=jax.ShapeDtypeStruct((NC * N_P, D), jnp.float32),
    mesh=plsc.VectorSubcoreMesh(core_axis_name="c", subcore_axis_name="s",
                                num_cores=NC, num_subcores=NS),
    scratch_types=[
        pltpu.VMEM((RS, 2, CH), jnp.int32),
        pltpu.VMEM((NBUF, CH, D), jnp.float32),
        pltpu.VMEM_SHARED((N_P, D), jnp.float32),
        pltpu.SemaphoreType.DMA((RS,)),
        pltpu.SemaphoreType.DMA((NBUF,)),
    ],
    name="sc_spmm",
)


def _prep_body(degp_ref, x_ref, dinv_ref, s0_ref):
    deg = degp_ref[0] + degp_ref[1]
    dinv = lax.rsqrt(jnp.maximum(deg, 1.0))
    dinv_ref[...] = dinv
    s0_ref[...] = dinv * x_ref[...]


def _prep(degp, x_pad):
    grid = N_P // BLK
    return pl.pallas_call(
        _prep_body,
        grid=(grid,),
        in_specs=[
            pl.BlockSpec((2, BLK, D), lambda b: (0, b, 0)),
            pl.BlockSpec((BLK, D), lambda b: (b, 0)),
        ],
        out_specs=[
            pl.BlockSpec((BLK, D), lambda b: (b, 0)),
            pl.BlockSpec((BLK, D), lambda b: (b, 0)),
        ],
        out_shape=[
            jax.ShapeDtypeStruct((N_P, D), jnp.float32),
            jax.ShapeDtypeStruct((N_P, D), jnp.float32),
        ],
    )(degp, x_pad)


def _mid_body(p_ref, dinv_ref, s1_ref):
    dinv = dinv_ref[...]
    s1_ref[...] = dinv * dinv * (p_ref[0] + p_ref[1])


def _mid(p, dinv):
    grid = N_P // BLK
    return pl.pallas_call(
        _mid_body,
        grid=(grid,),
        in_specs=[
            pl.BlockSpec((2, BLK, D), lambda b: (0, b, 0)),
            pl.BlockSpec((BLK, D), lambda b: (b, 0)),
        ],
        out_specs=pl.BlockSpec((BLK, D), lambda b: (b, 0)),
        out_shape=jax.ShapeDtypeStruct((N_P, D), jnp.float32),
    )(p, dinv)


def _end_body(h_ref, s1_ref, p2_ref, dinv_ref, w_ref, g_ref, b_ref,
              skip_ref, h_new_ref, s_next_ref, *, od, has_skip):
    dinv = dinv_ref[...]
    f1 = s1_ref[...] / dinv
    f2 = dinv * (p2_ref[0] + p2_ref[1])
    w = w_ref[...]
    pre = jnp.dot(h_ref[...], w[:D], preferred_element_type=jnp.float32)
    pre += jnp.dot(f1, w[D:2 * D], preferred_element_type=jnp.float32)
    pre += jnp.dot(f2, w[2 * D:], preferred_element_type=jnp.float32)
    h_new = jnp.maximum(g_ref[...] * pre + b_ref[...], 0.0)
    if has_skip:
        h_new = h_new + skip_ref[...]
    h_new_ref[...] = h_new
    s_next_ref[...] = dinv[:, :od] * h_new


def _end(h, s1, p2, dinv, w, gamma, beta, skip):
    od = w.shape[1]
    has_skip = skip is not None
    if skip is None:
        skip = jnp.zeros((1, 1), jnp.float32)
        skip_spec = pl.BlockSpec((1, 1), lambda b: (0, 0))
    else:
        skip_spec = pl.BlockSpec((BLK, od), lambda b: (b, 0))
    grid = N_P // BLK
    body = functools.partial(_end_body, od=od, has_skip=has_skip)
    return pl.pallas_call(
        body,
        grid=(grid,),
        in_specs=[
            pl.BlockSpec((BLK, D), lambda b: (b, 0)),
            pl.BlockSpec((BLK, D), lambda b: (b, 0)),
            pl.BlockSpec((2, BLK, D), lambda b: (0, b, 0)),
            pl.BlockSpec((BLK, D), lambda b: (b, 0)),
            pl.BlockSpec((3 * D, od), lambda b: (0, 0)),
            pl.BlockSpec((1, od), lambda b: (0, 0)),
            pl.BlockSpec((1, od), lambda b: (0, 0)),
            skip_spec,
        ],
        out_specs=[
            pl.BlockSpec((BLK, od), lambda b: (b, 0)),
            pl.BlockSpec((BLK, od), lambda b: (b, 0)),
        ],
        out_shape=[
            jax.ShapeDtypeStruct((N_P, od), jnp.float32),
            jax.ShapeDtypeStruct((N_P, od), jnp.float32),
        ],
    )(h, s1, p2, dinv, w, gamma.reshape(1, od), beta.reshape(1, od), skip)


def _pool_body(h_ref, ag_ref, ab_ref, wo_ref, bo_ref, out_ref):
    h = h_ref[...]
    rows = lax.broadcasted_iota(jnp.int32, h.shape, 0)
    valid = rows < N
    neg = jnp.float32(-3.4e38)
    pos = jnp.float32(3.4e38)
    mx = jnp.max(jnp.where(valid, h, neg), axis=0)
    mn = jnp.min(jnp.where(valid, h, pos), axis=0)
    hz = jnp.where(valid, h, 0.0)
    sm = jnp.sum(hz, axis=0)
    mean = sm / N
    cent = jnp.where(valid, h - mean[None, :], 0.0)
    var = jnp.sum(cent * cent, axis=0) / (N - 1)
    std = jnp.sqrt(var)
    flat = jnp.concatenate([mx, mn, mean, std], axis=0)
    z = flat * ag_ref[0] + ab_ref[0]
    res = jnp.sum(z * wo_ref[:, 0]) + bo_ref[0, 0]
    out_ref[...] = jnp.full((1, 128), res, jnp.float32)


def _pool(h5, agg_gamma, agg_beta, w_out, b_out):
    return pl.pallas_call(
        _pool_body,
        out_shape=jax.ShapeDtypeStruct((1, 128), jnp.float32),
    )(h5, agg_gamma.reshape(1, -1), agg_beta.reshape(1, -1),
      w_out, b_out.reshape(1, 1))


def kernel(x, edge_index, Ws, gammas, betas, agg_gamma, agg_beta, W_out, b_out):
    src = edge_index[0]
    dst = edge_index[1]
    pad = jnp.full((E_PAD - E,), PAD_ROW, jnp.int32)
    srcp = jnp.concatenate([src, pad]).reshape(NW, NCH, CH)
    dstp = jnp.concatenate([dst, pad]).reshape(NW, NCH, CH)
    ei = jnp.stack([srcp, dstp], axis=2)
    x_pad = jnp.pad(x, ((0, N_P - N), (0, 0)))
    ones = jnp.ones((N_P, D), jnp.float32)

    spmm = lambda s: _spmm(ei, s).reshape(NC, N_P, D)

    degp = spmm(ones)
    dinv, s = _prep(degp, x_pad)

    h = x_pad
    outs = []
    for i in range(len(LAYERS)):
        p1 = spmm(s)
        s1 = _mid(p1, dinv)
        p2 = spmm(s1)
        skip = outs[i - 2] if (i >= 2 and LAYERS[i] == LAYERS[i - 2]) else None
        h, s = _end(h, s1, p2, dinv, Ws[i], gammas[i], betas[i], skip)
        outs.append(h)

    out = _pool(h, agg_gamma, agg_beta, W_out, b_out)
    return out[0, :1]
